# Initial kernel scaffold; baseline (speedup 1.0000x reference)
import jax
import jax.numpy as jnp
from jax.experimental import pallas as pl


def kernel(height_ids, width_ids, h_embed, w_embed):
    raise NotImplementedError("write your pallas kernel here")


# speedup vs baseline: 4.1297x; 4.1297x over previous
import functools

import jax
import jax.numpy as jnp
from jax import lax
from jax.experimental import pallas as pl
from jax.experimental.pallas import tpu as pltpu
from jax.experimental.pallas import tpu_sc as plsc

_DIM = 768
_MAX_H = 32
_MAX_W = 32
_B = 64
_L = 1024
_N = _B * _L

_NC = 2
_NS = 16
_NW = _NC * _NS
_BPW = _N // _NW
_CH = 64
_NCH = _BPW // _CH


def _prep_body(hid_ref, wid_ref, h_ref, w_ref, cid_ref, tab_ref):
    cid_ref[...] = hid_ref[...] * _MAX_W + wid_ref[...]
    tab_ref[...] = h_ref[...][:, None, :] + w_ref[...][None, :, :]


def _prep(height_ids, width_ids, h_embed, w_embed):
    return pl.pallas_call(
        _prep_body,
        out_shape=[
            jax.ShapeDtypeStruct((_B, _L), jnp.int32),
            jax.ShapeDtypeStruct((_MAX_H, _MAX_W, _DIM), jnp.float32),
        ],
    )(height_ids, width_ids, h_embed, w_embed)


_sc_mesh = plsc.VectorSubcoreMesh(
    core_axis_name="c", subcore_axis_name="s", num_cores=_NC, num_subcores=_NS
)


@functools.partial(
    pl.kernel,
    mesh=_sc_mesh,
    out_type=jax.ShapeDtypeStruct((_N, _DIM), jnp.float32),
    scratch_types=[
        pltpu.VMEM((_BPW,), jnp.int32),
        pltpu.VMEM((_CH, _DIM), jnp.float32),
        pltpu.SemaphoreType.DMA,
    ],
)
def _sc_gather(tab_hbm, cid_hbm, out_hbm, idx_v, buf, gsem):
    wid = lax.axis_index("s") * _NC + lax.axis_index("c")
    base = wid * _BPW
    pltpu.sync_copy(cid_hbm.at[pl.ds(base, _BPW)], idx_v)

    def chunk(c, carry):
        off = c * _CH
        pltpu.async_copy(tab_hbm.at[idx_v.at[pl.ds(off, _CH)]], buf, gsem).wait()
        pltpu.sync_copy(buf, out_hbm.at[pl.ds(base + off, _CH)])
        return carry

    lax.fori_loop(0, _NCH, chunk, 0)


def kernel(height_ids, width_ids, h_embed, w_embed):
    hid = height_ids.astype(jnp.int32)
    wid = width_ids.astype(jnp.int32)
    cid, tab = _prep(hid, wid, h_embed, w_embed)
    tab = tab.reshape(_MAX_H * _MAX_W, _DIM)
    out = _sc_gather(tab, cid.reshape(_N))
    return out.reshape(_B, _L, _DIM)


# speedup vs baseline: 4.4636x; 1.0809x over previous
import functools

import jax
import jax.numpy as jnp
from jax import lax
from jax.experimental import pallas as pl
from jax.experimental.pallas import tpu as pltpu
from jax.experimental.pallas import tpu_sc as plsc

_DIM = 768
_MAX_H = 32
_MAX_W = 32
_B = 64
_L = 1024
_N = _B * _L

_NC = 2
_NS = 16
_NW = _NC * _NS
_BPW = _N // _NW
_CH = 64
_NCH = _BPW // _CH
_NG = _NCH // 2


def _prep_body(hid_ref, wid_ref, h_ref, w_ref, cid_ref, tab_ref):
    cid_ref[...] = hid_ref[...] * _MAX_W + wid_ref[...]
    tab_ref[...] = h_ref[...][:, None, :] + w_ref[...][None, :, :]


def _prep(height_ids, width_ids, h_embed, w_embed):
    return pl.pallas_call(
        _prep_body,
        out_shape=[
            jax.ShapeDtypeStruct((_B, _L), jnp.int32),
            jax.ShapeDtypeStruct((_MAX_H, _MAX_W, _DIM), jnp.float32),
        ],
    )(height_ids, width_ids, h_embed, w_embed)


_sc_mesh = plsc.VectorSubcoreMesh(
    core_axis_name="c", subcore_axis_name="s", num_cores=_NC, num_subcores=_NS
)


@functools.partial(
    pl.kernel,
    mesh=_sc_mesh,
    out_type=jax.ShapeDtypeStruct((_N, _DIM), jnp.float32),
    scratch_types=[
        pltpu.VMEM((_BPW,), jnp.int32),
        pltpu.VMEM((_CH, _DIM), jnp.float32),
        pltpu.VMEM((_CH, _DIM), jnp.float32),
        pltpu.SemaphoreType.DMA,
        pltpu.SemaphoreType.DMA,
        pltpu.SemaphoreType.DMA,
        pltpu.SemaphoreType.DMA,
    ],
)
def _sc_gather(tab_hbm, cid_hbm, out_hbm, idx_v, buf0, buf1, gs0, gs1, ss0, ss1):
    wid = lax.axis_index("s") * _NC + lax.axis_index("c")
    base = wid * _BPW
    pltpu.sync_copy(cid_hbm.at[pl.ds(base, _BPW)], idx_v)

    def g_src(c):
        return tab_hbm.at[idx_v.at[pl.ds(c * _CH, _CH)]]

    def o_dst(c):
        return out_hbm.at[pl.ds(base + c * _CH, _CH)]

    pltpu.async_copy(g_src(0), buf0, gs0)

    def body(g, carry):
        c0 = 2 * g
        c1 = c0 + 1
        pltpu.make_async_copy(g_src(c0), buf0, gs0).wait()

        @pl.when(g > 0)
        def _():
            pltpu.make_async_copy(buf1, o_dst(c1 - 2), ss1).wait()

        pltpu.async_copy(g_src(c1), buf1, gs1)
        pltpu.async_copy(buf0, o_dst(c0), ss0)
        pltpu.make_async_copy(g_src(c1), buf1, gs1).wait()

        @pl.when(g < _NG - 1)
        def _():
            pltpu.make_async_copy(buf0, o_dst(c0), ss0).wait()
            pltpu.async_copy(g_src(c0 + 2), buf0, gs0)

        pltpu.async_copy(buf1, o_dst(c1), ss1)
        return carry

    lax.fori_loop(0, _NG, body, 0)
    pltpu.make_async_copy(buf0, o_dst(_NCH - 2), ss0).wait()
    pltpu.make_async_copy(buf1, o_dst(_NCH - 1), ss1).wait()


def kernel(height_ids, width_ids, h_embed, w_embed):
    hid = height_ids.astype(jnp.int32)
    wid = width_ids.astype(jnp.int32)
    cid, tab = _prep(hid, wid, h_embed, w_embed)
    tab = tab.reshape(_MAX_H * _MAX_W, _DIM)
    out = _sc_gather(tab, cid.reshape(_N))
    return out.reshape(_B, _L, _DIM)


# speedup vs baseline: 4.5337x; 1.0157x over previous
import functools

import jax
import jax.numpy as jnp
from jax import lax
from jax.experimental import pallas as pl
from jax.experimental.pallas import tpu as pltpu
from jax.experimental.pallas import tpu_sc as plsc

_DIM = 768
_MAX_H = 32
_MAX_W = 32
_B = 64
_L = 1024
_N = _B * _L

_NC = 2
_NS = 16
_NW = _NC * _NS
_BPW = _N // _NW
_CH = 32
_NCH = _BPW // _CH
_NBUF = 4
_NG = _NCH // _NBUF


def _prep_body(hid_ref, wid_ref, h_ref, w_ref, cid_ref, tab_ref):
    cid_ref[...] = hid_ref[...] * _MAX_W + wid_ref[...]
    tab_ref[...] = h_ref[...][:, None, :] + w_ref[...][None, :, :]


def _prep(height_ids, width_ids, h_embed, w_embed):
    return pl.pallas_call(
        _prep_body,
        out_shape=[
            jax.ShapeDtypeStruct((_B, _L), jnp.int32),
            jax.ShapeDtypeStruct((_MAX_H, _MAX_W, _DIM), jnp.float32),
        ],
    )(height_ids, width_ids, h_embed, w_embed)


_sc_mesh = plsc.VectorSubcoreMesh(
    core_axis_name="c", subcore_axis_name="s", num_cores=_NC, num_subcores=_NS
)


@functools.partial(
    pl.kernel,
    mesh=_sc_mesh,
    out_type=jax.ShapeDtypeStruct((_N, _DIM), jnp.float32),
    scratch_types=[
        pltpu.VMEM((_BPW,), jnp.int32),
        [pltpu.VMEM((_CH, _DIM), jnp.float32) for _ in range(_NBUF)],
        [pltpu.SemaphoreType.DMA for _ in range(_NBUF)],
        [pltpu.SemaphoreType.DMA for _ in range(_NBUF)],
    ],
)
def _sc_gather(tab_hbm, cid_hbm, out_hbm, idx_v, bufs, gs, ss):
    wid = lax.axis_index("s") * _NC + lax.axis_index("c")
    base = wid * _BPW
    pltpu.sync_copy(cid_hbm.at[pl.ds(base, _BPW)], idx_v)

    def g_src(c):
        return tab_hbm.at[idx_v.at[pl.ds(c * _CH, _CH)]]

    def o_dst(c):
        return out_hbm.at[pl.ds(base + c * _CH, _CH)]

    pltpu.async_copy(g_src(0), bufs[0], gs[0])
    pltpu.async_copy(g_src(1), bufs[1], gs[1])

    def body(g, carry):
        for b in range(_NBUF):
            c = _NBUF * g + b
            b2 = (b + 2) % _NBUF
            if b < 2:
                @pl.when(g > 0)
                def _(b2=b2, c=c):
                    pltpu.make_async_copy(bufs[b2], o_dst(c - 2), ss[b2]).wait()

                pltpu.async_copy(g_src(c + 2), bufs[b2], gs[b2])
            else:
                pltpu.make_async_copy(bufs[b2], o_dst(c - 2), ss[b2]).wait()

                @pl.when(g < _NG - 1)
                def _(b2=b2, c=c):
                    pltpu.async_copy(g_src(c + 2), bufs[b2], gs[b2])

            pltpu.make_async_copy(g_src(c), bufs[b], gs[b]).wait()
            pltpu.async_copy(bufs[b], o_dst(c), ss[b])
        return carry

    lax.fori_loop(0, _NG, body, 0)
    pltpu.make_async_copy(bufs[2], o_dst(_NCH - 2), ss[2]).wait()
    pltpu.make_async_copy(bufs[3], o_dst(_NCH - 1), ss[3]).wait()


def kernel(height_ids, width_ids, h_embed, w_embed):
    hid = height_ids.astype(jnp.int32)
    wid = width_ids.astype(jnp.int32)
    cid, tab = _prep(hid, wid, h_embed, w_embed)
    tab = tab.reshape(_MAX_H * _MAX_W, _DIM)
    out = _sc_gather(tab, cid.reshape(_N))
    return out.reshape(_B, _L, _DIM)
